# baseline (speedup 1.0000x reference)
import functools

import numpy as np
import jax
import jax.numpy as jnp
from jax.experimental import pallas as pl
from jax.experimental.pallas import tpu as pltpu

_K = 9
_NA = 5
_NH = 19
_NW = 19
_NPIX = _NH * _NW
_NB = 32
_NLBL = 2 * _K + 3
_NGT = 50
_CONF0 = float(np.exp(2.0) - 1.0 + 1e-5)
_ANCHORS = [1.482, 2.2412, 2.0501, 3.1265, 2.3946, 4.6891, 3.1018, 3.0157,
            4.5509, 5.9446]
_OBJ_SCALE_SQRT = float(np.sqrt(5.0))
_BPS = 2
_XCH = [0] + [k + 2 for k in range(1, _K)]
_YCH = [1] + [k + 3 for k in range(1, _K)]


def _sig(x):
    return 1.0 / (1.0 + jnp.exp(-x))


def _one_batch(cur, prev, tgt):
    f32 = jnp.float32
    i32 = jnp.int32

    pixi = jax.lax.broadcasted_iota(i32, (1, _NPIX), 1)
    gxpix = (pixi % _NW).astype(f32)
    gypix = (pixi // _NW).astype(f32)

    gx = [tgt[:, 1 + 2 * k:2 + 2 * k] for k in range(_K)]
    gy = [tgt[:, 2 + 2 * k:3 + 2 * k] for k in range(_K)]

    ti = jax.lax.broadcasted_iota(i32, (_NGT, _NGT), 0)
    si = jax.lax.broadcasted_iota(i32, (_NGT, _NGT), 1)
    tril = (si <= ti).astype(f32)
    ind0 = (tgt[:, 1:2] == 0.0).astype(f32)
    zcnt = jax.lax.dot_general(tril, ind0, (((1,), (0,)), ((), ())),
                               preferred_element_type=f32,
                               precision=jax.lax.Precision.HIGHEST)
    validf = (zcnt == 0.0).astype(f32)

    gi0 = (gx[0] * float(_NW)).astype(i32)
    gj0 = (gy[0] * float(_NH)).astype(i32)
    gi0f = gi0.astype(f32)
    gj0f = gj0.astype(f32)
    q = gj0 * _NW + gi0
    qpos = jax.lax.broadcasted_iota(i32, (_NGT, _NPIX), 1)
    ohq = (qpos == q).astype(f32)

    gw = tgt[:, _NLBL - 2:_NLBL - 1] * float(_NW)
    gh = tgt[:, _NLBL - 1:_NLBL] * float(_NH)
    ious = []
    for n in range(_NA):
        aw = _ANCHORS[2 * n]
        ah = _ANCHORS[2 * n + 1]
        mx = jnp.minimum(-aw / 2.0, -gw / 2.0)
        Mx = jnp.maximum(aw / 2.0, gw / 2.0)
        my = jnp.minimum(-ah / 2.0, -gh / 2.0)
        My = jnp.maximum(ah / 2.0, gh / 2.0)
        cw = aw + gw - (Mx - mx)
        chh = ah + gh - (My - my)
        carea = cw * chh
        uarea = aw * ah + gw * gh - carea
        ious.append(jnp.where((cw <= 0.0) | (chh <= 0.0), 0.0, carea / uarea))
    iouc = jnp.concatenate(ious, axis=1)
    best = jnp.max(iouc, axis=1, keepdims=True)
    nio = jax.lax.broadcasted_iota(i32, (_NGT, _NA), 1)
    first = jnp.min(jnp.where(iouc == best, nio, _NA + 1), axis=1,
                    keepdims=True)
    bn = jnp.where(best > 0.0, first, _NA - 1)

    c2 = 0.025 * float(np.log2(np.e))
    c1 = 2.0 * float(np.log2(np.e))
    thr_raw = 0.6 * 9.0 * _CONF0
    gxs = [g * (640.0 * c2) for g in gx]
    gys = [g * (480.0 * c2) for g in gy]
    noobj_vec = jnp.zeros((1, _NPIX), f32)
    G = jnp.zeros((_NGT, 14), f32)
    for a in range(_NA):
        acc = jnp.zeros((_NGT, _NPIX), f32)
        for k in range(_K):
            vx = cur[a * 32 + _XCH[k]:a * 32 + _XCH[k] + 1, :]
            vy = cur[a * 32 + _YCH[k]:a * 32 + _YCH[k] + 1, :]
            if k == 0:
                vx = _sig(vx)
                vy = _sig(vy)
            hx = (vx + gxpix) * (640.0 * c2 / float(_NW))
            hy = (vy + gypix) * (480.0 * c2 / float(_NH))
            dx = gxs[k] - hx
            dy = gys[k] - hy
            d2 = jnp.maximum(dx * dx + dy * dy, 1e-24)
            arg = c1 - d2 * jax.lax.rsqrt(d2)
            acc = acc + jnp.maximum(jnp.exp2(arg) - 1.0, 0.0)
        mc = jnp.max(acc * validf, axis=0, keepdims=True)
        confp_a = _sig(cur[a * 32 + 2 * _K:a * 32 + 2 * _K + 1, :])
        noobj_a = (mc <= thr_raw).astype(f32)
        noobj_vec = noobj_vec + noobj_a * confp_a * confp_a

        feat = jnp.concatenate(
            [cur[a * 32:a * 32 + 12, :],
             cur[a * 32 + 2 * _K:a * 32 + 2 * _K + 1, :],
             mc], axis=0)
        Ga = jax.lax.dot_general(ohq, feat, (((1,), (1,)), ((), ())),
                                 preferred_element_type=f32,
                               precision=jax.lax.Precision.HIGHEST)
        G = G + jnp.where(bn == a, 1.0, 0.0) * Ga

    noobj_sum = jnp.sum(noobj_vec)

    P = jax.lax.dot_general(ohq, prev,
                            (((1,), (1,)), ((), ())),
                            preferred_element_type=f32,
                               precision=jax.lax.Precision.HIGHEST)
    tacc = jnp.zeros((_NGT, 1), f32)
    for k in range(_K):
        vx = P[:, _XCH[k]:_XCH[k] + 1]
        vy = P[:, _YCH[k]:_YCH[k] + 1]
        if k == 0:
            vx = _sig(vx)
            vy = _sig(vy)
        pbx = (vx + gi0f) / float(_NW)
        pby = (vy + gj0f) / float(_NH)
        dxk = (gx[k] - pbx) * 640.0
        dyk = (gy[k] - pby) * 480.0
        dk = jnp.sqrt(dxk * dxk + dyk * dyk)
        tacc = tacc + jnp.maximum(jnp.exp(2.0 - dk * 0.025) - 1.0, 0.0)
    tconf = tacc * (1.0 / (9.0 * _CONF0))

    cellf = (bn * _NPIX + q).astype(f32)
    eyef = (ti == si).astype(f32)
    cell_row = jax.lax.dot_general(cellf, eyef, (((0,), (0,)), ((), ())),
                                   preferred_element_type=f32,
                               precision=jax.lax.Precision.HIGHEST)
    later_same = ((cellf == cell_row) & (si > ti)).astype(f32)
    kcnt = jax.lax.dot_general(later_same, validf, (((1,), (0,)), ((), ())),
                               preferred_element_type=f32,
                               precision=jax.lax.Precision.HIGHEST)
    winf = validf * (kcnt == 0.0).astype(f32)

    confp_c = _sig(G[:, 12:13])
    noobj_c = (G[:, 13:14] <= thr_raw).astype(f32)
    s5 = _OBJ_SCALE_SQRT
    dconf = confp_c * s5 - tconf * s5
    adj = 0.5 * (dconf * dconf - noobj_c * confp_c * confp_c)
    for k in range(_K):
        xc = G[:, _XCH[k]:_XCH[k] + 1]
        yc = G[:, _YCH[k]:_YCH[k] + 1]
        if k == 0:
            xc = _sig(xc)
            yc = _sig(yc)
        tx = gx[k] * float(_NW) - gi0f
        ty = gy[k] * float(_NH) - gj0f
        adj = adj + 0.5 * ((xc - tx) * (xc - tx) + (yc - ty) * (yc - ty))

    return 0.5 * noobj_sum + jnp.sum(winf * adj)


def _loss_body(cur_ref, prev_ref, tgt_ref, out_ref):
    g = pl.program_id(0)
    loss = _one_batch(cur_ref[0], prev_ref[0], tgt_ref[0])
    for j in range(1, _BPS):
        loss = loss + _one_batch(cur_ref[j], prev_ref[j], tgt_ref[j])

    @pl.when(g == 0)
    def _():
        out_ref[0, 0] = 0.0

    out_ref[0, 0] = out_ref[0, 0] + loss


@functools.partial(jax.jit, static_argnames=("interpret",))
def _region_loss(output, target, interpret=False):
    out_r = output.astype(jnp.float32).reshape(_NB, 160, _NPIX)
    prev12 = jnp.roll(out_r[:, 4 * 32:4 * 32 + 12, :], 1, axis=0)
    tgt_r = target.astype(jnp.float32).reshape(_NB, _NGT, _NLBL)
    res = pl.pallas_call(
        _loss_body,
        grid=(_NB // _BPS,),
        in_specs=[
            pl.BlockSpec((_BPS, 160, _NPIX), lambda g: (g, 0, 0)),
            pl.BlockSpec((_BPS, 12, _NPIX), lambda g: (g, 0, 0)),
            pl.BlockSpec((_BPS, _NGT, _NLBL), lambda g: (g, 0, 0)),
        ],
        out_specs=pl.BlockSpec((1, 1), lambda g: (0, 0),
                               memory_space=pltpu.SMEM),
        out_shape=jax.ShapeDtypeStruct((1, 1), jnp.float32),
        interpret=interpret,
    )(out_r, prev12, tgt_r)
    return res[0, 0]


def kernel(output, target, epoch):
    return _region_loss(output, target)


# speedup vs baseline: 1.1776x; 1.1776x over previous
import functools

import numpy as np
import jax
import jax.numpy as jnp
from jax.experimental import pallas as pl
from jax.experimental.pallas import tpu as pltpu

_K = 9
_NA = 5
_NH = 19
_NW = 19
_NPIX = _NH * _NW
_NB = 32
_NLBL = 2 * _K + 3
_NGT = 50
_CONF0 = float(np.exp(2.0) - 1.0 + 1e-5)
_ANCHORS = [1.482, 2.2412, 2.0501, 3.1265, 2.3946, 4.6891, 3.1018, 3.0157,
            4.5509, 5.9446]
_OBJ_SCALE_SQRT = float(np.sqrt(5.0))
_BPS = 1
_XCH = [0] + [k + 2 for k in range(1, _K)]
_YCH = [1] + [k + 3 for k in range(1, _K)]


def _sig(x):
    return 1.0 / (1.0 + jnp.exp(-x))


def _one_batch(cur, prev, tgt):
    f32 = jnp.float32
    i32 = jnp.int32

    pixi = jax.lax.broadcasted_iota(i32, (1, _NPIX), 1)
    gxpix = (pixi % _NW).astype(f32)
    gypix = (pixi // _NW).astype(f32)

    gx = [tgt[:, 1 + 2 * k:2 + 2 * k] for k in range(_K)]
    gy = [tgt[:, 2 + 2 * k:3 + 2 * k] for k in range(_K)]

    ti = jax.lax.broadcasted_iota(i32, (_NGT, _NGT), 0)
    si = jax.lax.broadcasted_iota(i32, (_NGT, _NGT), 1)
    tril = (si <= ti).astype(f32)
    ind0 = (tgt[:, 1:2] == 0.0).astype(f32)
    zcnt = jax.lax.dot_general(tril, ind0, (((1,), (0,)), ((), ())),
                               preferred_element_type=f32,
                               precision=jax.lax.Precision.HIGHEST)
    validf = (zcnt == 0.0).astype(f32)

    gi0 = (gx[0] * float(_NW)).astype(i32)
    gj0 = (gy[0] * float(_NH)).astype(i32)
    gi0f = gi0.astype(f32)
    gj0f = gj0.astype(f32)
    q = gj0 * _NW + gi0
    qpos = jax.lax.broadcasted_iota(i32, (_NGT, _NPIX), 1)
    ohq = (qpos == q).astype(f32)

    gw = tgt[:, _NLBL - 2:_NLBL - 1] * float(_NW)
    gh = tgt[:, _NLBL - 1:_NLBL] * float(_NH)
    ious = []
    for n in range(_NA):
        aw = _ANCHORS[2 * n]
        ah = _ANCHORS[2 * n + 1]
        mx = jnp.minimum(-aw / 2.0, -gw / 2.0)
        Mx = jnp.maximum(aw / 2.0, gw / 2.0)
        my = jnp.minimum(-ah / 2.0, -gh / 2.0)
        My = jnp.maximum(ah / 2.0, gh / 2.0)
        cw = aw + gw - (Mx - mx)
        chh = ah + gh - (My - my)
        carea = cw * chh
        uarea = aw * ah + gw * gh - carea
        ious.append(jnp.where((cw <= 0.0) | (chh <= 0.0), 0.0, carea / uarea))
    iouc = jnp.concatenate(ious, axis=1)
    best = jnp.max(iouc, axis=1, keepdims=True)
    nio = jax.lax.broadcasted_iota(i32, (_NGT, _NA), 1)
    first = jnp.min(jnp.where(iouc == best, nio, _NA + 1), axis=1,
                    keepdims=True)
    bn = jnp.where(best > 0.0, first, _NA - 1)

    c2 = 0.025 * float(np.log2(np.e))
    c1 = 2.0 * float(np.log2(np.e))
    thr_raw = 0.6 * 9.0 * _CONF0
    bf = jnp.bfloat16
    gxs = [(g * (640.0 * c2)).astype(bf) for g in gx]
    gys = [(g * (480.0 * c2)).astype(bf) for g in gy]
    noobj_vec = jnp.zeros((1, _NPIX), f32)
    G = jnp.zeros((_NGT, 14), f32)
    for a in range(_NA):
        acc = jnp.zeros((_NGT, _NPIX), bf)
        for k in range(_K):
            vx = cur[a * 32 + _XCH[k]:a * 32 + _XCH[k] + 1, :]
            vy = cur[a * 32 + _YCH[k]:a * 32 + _YCH[k] + 1, :]
            if k == 0:
                vx = _sig(vx)
                vy = _sig(vy)
            hx = ((vx + gxpix) * (640.0 * c2 / float(_NW))).astype(bf)
            hy = ((vy + gypix) * (480.0 * c2 / float(_NH))).astype(bf)
            dx = gxs[k] - hx
            dy = gys[k] - hy
            d2 = jnp.clip(dx * dx + dy * dy, bf(1e-24), bf(1e30))
            arg = bf(c1) - d2 * jax.lax.rsqrt(d2)
            acc = acc + jnp.maximum(jnp.exp2(arg) - bf(1.0), bf(0.0))
        mc = jnp.max(acc.astype(f32) * validf, axis=0, keepdims=True)
        confp_a = _sig(cur[a * 32 + 2 * _K:a * 32 + 2 * _K + 1, :])
        noobj_a = (mc <= thr_raw).astype(f32)
        noobj_vec = noobj_vec + noobj_a * confp_a * confp_a

        feat = jnp.concatenate(
            [cur[a * 32:a * 32 + 12, :],
             cur[a * 32 + 2 * _K:a * 32 + 2 * _K + 1, :],
             mc], axis=0)
        Ga = jax.lax.dot_general(ohq, feat, (((1,), (1,)), ((), ())),
                                 preferred_element_type=f32,
                               precision=jax.lax.Precision.HIGHEST)
        G = G + jnp.where(bn == a, 1.0, 0.0) * Ga

    noobj_sum = jnp.sum(noobj_vec)

    P = jax.lax.dot_general(ohq, prev,
                            (((1,), (1,)), ((), ())),
                            preferred_element_type=f32,
                               precision=jax.lax.Precision.HIGHEST)
    tacc = jnp.zeros((_NGT, 1), f32)
    for k in range(_K):
        vx = P[:, _XCH[k]:_XCH[k] + 1]
        vy = P[:, _YCH[k]:_YCH[k] + 1]
        if k == 0:
            vx = _sig(vx)
            vy = _sig(vy)
        pbx = (vx + gi0f) / float(_NW)
        pby = (vy + gj0f) / float(_NH)
        dxk = (gx[k] - pbx) * 640.0
        dyk = (gy[k] - pby) * 480.0
        dk = jnp.sqrt(dxk * dxk + dyk * dyk)
        tacc = tacc + jnp.maximum(jnp.exp(2.0 - dk * 0.025) - 1.0, 0.0)
    tconf = tacc * (1.0 / (9.0 * _CONF0))

    cellf = (bn * _NPIX + q).astype(f32)
    eyef = (ti == si).astype(f32)
    cell_row = jax.lax.dot_general(cellf, eyef, (((0,), (0,)), ((), ())),
                                   preferred_element_type=f32,
                               precision=jax.lax.Precision.HIGHEST)
    later_same = ((cellf == cell_row) & (si > ti)).astype(f32)
    kcnt = jax.lax.dot_general(later_same, validf, (((1,), (0,)), ((), ())),
                               preferred_element_type=f32,
                               precision=jax.lax.Precision.HIGHEST)
    winf = validf * (kcnt == 0.0).astype(f32)

    confp_c = _sig(G[:, 12:13])
    noobj_c = (G[:, 13:14] <= thr_raw).astype(f32)
    s5 = _OBJ_SCALE_SQRT
    dconf = confp_c * s5 - tconf * s5
    adj = 0.5 * (dconf * dconf - noobj_c * confp_c * confp_c)
    for k in range(_K):
        xc = G[:, _XCH[k]:_XCH[k] + 1]
        yc = G[:, _YCH[k]:_YCH[k] + 1]
        if k == 0:
            xc = _sig(xc)
            yc = _sig(yc)
        tx = gx[k] * float(_NW) - gi0f
        ty = gy[k] * float(_NH) - gj0f
        adj = adj + 0.5 * ((xc - tx) * (xc - tx) + (yc - ty) * (yc - ty))

    return 0.5 * noobj_sum + jnp.sum(winf * adj)


def _loss_body(cur_ref, prev_ref, tgt_ref, out_ref):
    g = pl.program_id(0)
    loss = _one_batch(cur_ref[0], prev_ref[0], tgt_ref[0])
    for j in range(1, _BPS):
        loss = loss + _one_batch(cur_ref[j], prev_ref[j], tgt_ref[j])

    @pl.when(g == 0)
    def _():
        out_ref[0, 0] = 0.0

    out_ref[0, 0] = out_ref[0, 0] + loss


@functools.partial(jax.jit, static_argnames=("interpret",))
def _region_loss(output, target, interpret=False):
    out_r = output.astype(jnp.float32).reshape(_NB, 160, _NPIX)
    prev12 = jnp.roll(out_r[:, 4 * 32:4 * 32 + 12, :], 1, axis=0)
    tgt_r = target.astype(jnp.float32).reshape(_NB, _NGT, _NLBL)
    res = pl.pallas_call(
        _loss_body,
        grid=(_NB // _BPS,),
        in_specs=[
            pl.BlockSpec((_BPS, 160, _NPIX), lambda g: (g, 0, 0)),
            pl.BlockSpec((_BPS, 12, _NPIX), lambda g: (g, 0, 0)),
            pl.BlockSpec((_BPS, _NGT, _NLBL), lambda g: (g, 0, 0)),
        ],
        out_specs=pl.BlockSpec((1, 1), lambda g: (0, 0),
                               memory_space=pltpu.SMEM),
        out_shape=jax.ShapeDtypeStruct((1, 1), jnp.float32),
        interpret=interpret,
    )(out_r, prev12, tgt_r)
    return res[0, 0]


def kernel(output, target, epoch):
    return _region_loss(output, target)


# speedup vs baseline: 1.6279x; 1.3823x over previous
import functools

import numpy as np
import jax
import jax.numpy as jnp
from jax.experimental import pallas as pl
from jax.experimental.pallas import tpu as pltpu

_K = 9
_NA = 5
_NH = 19
_NW = 19
_NPIX = _NH * _NW
_NB = 32
_NLBL = 2 * _K + 3
_NGT = 50
_CONF0 = float(np.exp(2.0) - 1.0 + 1e-5)
_ANCHORS = [1.482, 2.2412, 2.0501, 3.1265, 2.3946, 4.6891, 3.1018, 3.0157,
            4.5509, 5.9446]
_OBJ_SCALE_SQRT = float(np.sqrt(5.0))
_XCH = [0] + [k + 2 for k in range(1, _K)]
_YCH = [1] + [k + 3 for k in range(1, _K)]


def _sig(x):
    return 1.0 / (1.0 + jnp.exp(-x))


def _one_batch(cur, prev, tgt, tgtT):
    f32 = jnp.float32
    i32 = jnp.int32
    bf = jnp.bfloat16

    pixi = jax.lax.broadcasted_iota(i32, (1, _NPIX), 1)
    gxpix = (pixi % _NW).astype(f32)
    gypix = (pixi // _NW).astype(f32)

    gxr = [tgtT[1 + 2 * k:2 + 2 * k, :] for k in range(_K)]
    gyr = [tgtT[2 + 2 * k:3 + 2 * k, :] for k in range(_K)]

    ti = jax.lax.broadcasted_iota(i32, (_NGT, _NGT), 0)
    si = jax.lax.broadcasted_iota(i32, (_NGT, _NGT), 1)
    eyef = (ti == si).astype(f32)

    ind0r = (tgtT[1:2, :] == 0.0).astype(f32)
    le = (ti <= si).astype(f32)
    cntr = jax.lax.dot_general(ind0r, le, (((1,), (0,)), ((), ())),
                               preferred_element_type=f32)
    validr = (cntr == 0.0).astype(f32)
    validf = jax.lax.dot_general(eyef, validr, (((1,), (1,)), ((), ())),
                                 preferred_element_type=f32)

    gi0r = (gxr[0] * float(_NW)).astype(i32)
    gj0r = (gyr[0] * float(_NH)).astype(i32)
    gi0fr = gi0r.astype(f32)
    gj0fr = gj0r.astype(f32)
    qr = gj0r * _NW + gi0r

    ohqT = (jax.lax.broadcasted_iota(i32, (_NPIX, _NGT), 0) == qr)
    ohqT = ohqT.astype(f32)

    gw = tgtT[_NLBL - 2:_NLBL - 1, :] * float(_NW)
    gh = tgtT[_NLBL - 1:_NLBL, :] * float(_NH)
    ious = []
    for n in range(_NA):
        aw = _ANCHORS[2 * n]
        ah = _ANCHORS[2 * n + 1]
        mx = jnp.minimum(-aw / 2.0, -gw / 2.0)
        Mx = jnp.maximum(aw / 2.0, gw / 2.0)
        my = jnp.minimum(-ah / 2.0, -gh / 2.0)
        My = jnp.maximum(ah / 2.0, gh / 2.0)
        cw = aw + gw - (Mx - mx)
        chh = ah + gh - (My - my)
        carea = cw * chh
        uarea = aw * ah + gw * gh - carea
        ious.append(jnp.where((cw <= 0.0) | (chh <= 0.0), 0.0, carea / uarea))
    iouc = jnp.concatenate(ious, axis=0)
    best = jnp.max(iouc, axis=0, keepdims=True)
    nio = jax.lax.broadcasted_iota(i32, (_NA, _NGT), 0)
    first = jnp.min(jnp.where(iouc == best, nio, _NA + 1), axis=0,
                    keepdims=True)
    bnr = jnp.where(best > 0.0, first, _NA - 1)

    c2 = 0.025 * float(np.log2(np.e))
    c1 = 2.0 * float(np.log2(np.e))
    sx = 640.0 * c2 / float(_NW)
    sy = 480.0 * c2 / float(_NH)
    thr_raw = 0.6 * 9.0 * _CONF0
    gxs = [(tgt[:, 1 + 2 * k:2 + 2 * k] * (640.0 * c2)).astype(bf)
           for k in range(_K)]
    gys = [(tgt[:, 2 + 2 * k:3 + 2 * k] * (480.0 * c2)).astype(bf)
           for k in range(_K)]
    noobj_vec = jnp.zeros((1, _NPIX), f32)
    GT = jnp.zeros((14, _NGT), f32)
    for a in range(_NA):
        r0 = _sig(cur[a * 32:a * 32 + 2, :])
        hx_all = ((jnp.concatenate([r0[0:1], cur[a * 32 + 3:a * 32 + 11, :]],
                                   axis=0) + gxpix) * sx).astype(bf)
        hy_all = ((jnp.concatenate([r0[1:2], cur[a * 32 + 4:a * 32 + 12, :]],
                                   axis=0) + gypix) * sy).astype(bf)
        acc = jnp.zeros((_NGT, _NPIX), bf)
        for k in range(_K):
            dx = gxs[k] - hx_all[k:k + 1, :]
            dy = gys[k] - hy_all[k:k + 1, :]
            d2 = jnp.maximum(dx * dx + dy * dy, bf(1e-24))
            arg = bf(c1) - d2 * jax.lax.rsqrt(d2)
            acc = acc + jnp.maximum(jnp.exp2(arg) - bf(1.0), bf(0.0))
        mc = jnp.max(acc.astype(f32) * validf, axis=0, keepdims=True)
        confp_a = _sig(cur[a * 32 + 2 * _K:a * 32 + 2 * _K + 1, :])
        noobj_a = (mc <= thr_raw).astype(f32)
        noobj_vec = noobj_vec + noobj_a * confp_a * confp_a

        feat = jnp.concatenate(
            [cur[a * 32:a * 32 + 12, :],
             cur[a * 32 + 2 * _K:a * 32 + 2 * _K + 1, :],
             mc], axis=0)
        GaT = jax.lax.dot_general(feat, ohqT, (((1,), (0,)), ((), ())),
                                  preferred_element_type=f32)
        GT = GT + jnp.where(bnr == a, 1.0, 0.0) * GaT

    noobj_sum = jnp.sum(noobj_vec)

    PT = jax.lax.dot_general(prev, ohqT, (((1,), (0,)), ((), ())),
                             preferred_element_type=f32)
    tacc = jnp.zeros((1, _NGT), f32)
    for k in range(_K):
        vx = PT[_XCH[k]:_XCH[k] + 1, :]
        vy = PT[_YCH[k]:_YCH[k] + 1, :]
        if k == 0:
            vx = _sig(vx)
            vy = _sig(vy)
        pbx = (vx + gi0fr) / float(_NW)
        pby = (vy + gj0fr) / float(_NH)
        dxk = (gxr[k] - pbx) * 640.0
        dyk = (gyr[k] - pby) * 480.0
        dk = jnp.sqrt(dxk * dxk + dyk * dyk)
        tacc = tacc + jnp.maximum(jnp.exp(2.0 - dk * 0.025) - 1.0, 0.0)
    tconf = tacc * (1.0 / (9.0 * _CONF0))

    cellr = (bnr * _NPIX + qr).astype(f32)
    cellc = jax.lax.dot_general(eyef, cellr, (((1,), (1,)), ((), ())),
                                preferred_element_type=f32,
                                precision=jax.lax.Precision.HIGHEST)
    later_same = ((cellc == cellr) & (ti > si)).astype(f32)
    kcntr = jax.lax.dot_general(validr, later_same,
                                (((1,), (0,)), ((), ())),
                                preferred_element_type=f32)
    winr = validr * (kcntr == 0.0).astype(f32)

    confp_c = _sig(GT[12:13, :])
    noobj_c = (GT[13:14, :] <= thr_raw).astype(f32)
    s5 = _OBJ_SCALE_SQRT
    dconf = confp_c * s5 - tconf * s5
    adj = 0.5 * (dconf * dconf - noobj_c * confp_c * confp_c)
    for k in range(_K):
        xc = GT[_XCH[k]:_XCH[k] + 1, :]
        yc = GT[_YCH[k]:_YCH[k] + 1, :]
        if k == 0:
            xc = _sig(xc)
            yc = _sig(yc)
        tx = gxr[k] * float(_NW) - gi0fr
        ty = gyr[k] * float(_NH) - gj0fr
        adj = adj + 0.5 * ((xc - tx) * (xc - tx) + (yc - ty) * (yc - ty))

    return 0.5 * noobj_sum + jnp.sum(winr * adj)


def _loss_body(cur_ref, prev_ref, tgt_ref, tgtT_ref, out_ref):
    g = pl.program_id(0)
    loss = _one_batch(cur_ref[0], prev_ref[0], tgt_ref[0], tgtT_ref[0])

    @pl.when(g == 0)
    def _():
        out_ref[0, 0] = 0.0

    out_ref[0, 0] = out_ref[0, 0] + loss


@functools.partial(jax.jit, static_argnames=("interpret",))
def _region_loss(output, target, interpret=False):
    out_r = output.astype(jnp.float32).reshape(_NB, 160, _NPIX)
    prev12 = jnp.roll(out_r[:, 4 * 32:4 * 32 + 12, :], 1, axis=0)
    tgt_r = target.astype(jnp.float32).reshape(_NB, _NGT, _NLBL)
    tgt_t = tgt_r.transpose(0, 2, 1)
    res = pl.pallas_call(
        _loss_body,
        grid=(_NB,),
        in_specs=[
            pl.BlockSpec((1, 160, _NPIX), lambda b: (b, 0, 0)),
            pl.BlockSpec((1, 12, _NPIX), lambda b: (b, 0, 0)),
            pl.BlockSpec((1, _NGT, _NLBL), lambda b: (b, 0, 0)),
            pl.BlockSpec((1, _NLBL, _NGT), lambda b: (b, 0, 0)),
        ],
        out_specs=pl.BlockSpec((1, 1), lambda b: (0, 0),
                               memory_space=pltpu.SMEM),
        out_shape=jax.ShapeDtypeStruct((1, 1), jnp.float32),
        interpret=interpret,
    )(out_r, prev12, tgt_r, tgt_t)
    return res[0, 0]


def kernel(output, target, epoch):
    return _region_loss(output, target)
